# baseline (speedup 1.0000x reference)
import functools

import jax
import jax.numpy as jnp
from jax import lax
from jax.experimental import pallas as pl
from jax.experimental.pallas import tpu as pltpu
from jax.experimental.pallas import tpu_sc as plsc

VOCAB = 1000000
EMBED_DIM = 64
BATCH = 16384

_info = plsc.get_sparse_core_info()
_NC, _NS = _info.num_cores, _info.num_subcores
_NW = _NC * _NS
_B_PER_W = BATCH // _NW


def _gather_body(tokens_hbm, table_hbm, out_hbm, idx_v, rows_v, sem):
    wid = lax.axis_index("s") * _NC + lax.axis_index("c")
    base = wid * _B_PER_W
    pltpu.sync_copy(tokens_hbm.at[pl.ds(base, _B_PER_W)], idx_v)
    pltpu.async_copy(table_hbm.at[idx_v], rows_v, sem).wait()
    pltpu.sync_copy(rows_v, out_hbm.at[pl.ds(base, _B_PER_W)])


@jax.jit
def _gather(tokens, W):
    mesh = plsc.VectorSubcoreMesh(core_axis_name="c", subcore_axis_name="s")
    return pl.kernel(
        _gather_body,
        out_type=jax.ShapeDtypeStruct((BATCH, EMBED_DIM), jnp.float32),
        mesh=mesh,
        scratch_types=[
            pltpu.VMEM((_B_PER_W,), jnp.int32),
            pltpu.VMEM((_B_PER_W, EMBED_DIM), jnp.float32),
            pltpu.SemaphoreType.DMA,
        ],
        compiler_params=pltpu.CompilerParams(use_tc_tiling_on_sc=False),
    )(tokens, W)


def kernel(tokens, values, W):
    del values
    return _gather(tokens, W)


# speedup vs baseline: 2.1457x; 2.1457x over previous
import functools

import jax
import jax.numpy as jnp
from jax import lax
from jax.experimental import pallas as pl
from jax.experimental.pallas import tpu as pltpu
from jax.experimental.pallas import tpu_sc as plsc

VOCAB = 1000000
EMBED_DIM = 64
BATCH = 16384

_info = plsc.get_sparse_core_info()
_NC, _NS, _NL = _info.num_cores, _info.num_subcores, _info.num_lanes
_NW = _NC * _NS

_CHUNK = 256
_NCHUNK = VOCAB // _CHUNK + 1
_LAST_LO = VOCAB - _CHUNK
_TOKPIECE = 2048
_NPIECE = BATCH // _TOKPIECE
_NROW = 16
_ROW_BYTES = EMBED_DIM * 4


def _gather_body(
    tokens_hbm,
    tableT_hbm,
    stage_hbm,
    tokpiece_v,
    my_tok_v,
    my_pos_v,
    hits_v,
    hpos_v,
    chunk_v,
    rows_v,
    cnt_smem,
    fired_smem,
    tsem,
    csem,
    rsem,
):
    wid = lax.axis_index("s") * _NC + lax.axis_index("c")
    cnt_smem[0] = 0
    fired_smem[0] = 0
    lanes = lax.iota(jnp.int32, _NL)

    def _piece_src(p):
        return tokens_hbm.at[pl.ds(p * _TOKPIECE, _TOKPIECE)]

    def _piece(p, pb):
        pltpu.sync_copy(_piece_src(p), tokpiece_v.at[pb])

        def _grp(j, _):
            t = tokpiece_v[pb, pl.ds(j * _NL, _NL)]
            g = jnp.minimum(lax.shift_right_logical(t, 8), _NCHUNK - 1)
            m = (g & (_NW - 1)) == wid
            cnt = cnt_smem[0]
            plsc.store_compressed(my_tok_v.at[pl.ds(cnt, _NL)], t, mask=m)
            pos = p * _TOKPIECE + j * _NL + lanes
            plsc.store_compressed(my_pos_v.at[pl.ds(cnt, _NL)], pos, mask=m)
            cnt_smem[0] = cnt + jnp.sum(m.astype(jnp.int32))
            return 0

        lax.fori_loop(0, _TOKPIECE // _NL, _grp, 0)

    for p in range(_NPIECE):
        _piece(p, 0)

    cnt = cnt_smem[0]
    ngrp = lax.div(cnt + _NL - 1, _NL)

    nk = lax.div(_NCHUNK - 1 - wid, _NW) + 1

    def _chunk_lo(g):
        return jnp.where(g == _NCHUNK - 1, _LAST_LO, g * _CHUNK)

    def _chunk_src(g):
        lo = pl.multiple_of(_chunk_lo(g), 128)
        return tableT_hbm.at[:, pl.ds(lo, _CHUNK)]

    def _start_chunk(k, cb):
        pltpu.async_copy(_chunk_src(wid + _NW * k), chunk_v.at[cb], csem)

    _start_chunk(0, 0)
    _start_chunk(1, 1)

    def _proc_chunk(k, cb):
        g = wid + _NW * k
        lo = _chunk_lo(g)
        pltpu.make_async_copy(_chunk_src(g), chunk_v.at[cb], csem).wait()

        def _grp(j, _):
            valid = (j * _NL + lanes) < cnt
            t = my_tok_v[pl.ds(j * _NL, _NL)]
            tg = jnp.minimum(lax.shift_right_logical(t, 8), _NCHUNK - 1)
            m = valid & (tg == g)

            @pl.when(jnp.any(m))
            def _():
                plsc.store_compressed(hits_v.at[...], t, mask=m)
                plsc.store_compressed(hpos_v.at[...], my_pos_v[pl.ds(j * _NL, _NL)], mask=m)
                nh = jnp.sum(m.astype(jnp.int32))

                def _tok(j2, _):
                    sel = jnp.full((_NL,), j2, jnp.int32)
                    tjv = plsc.load_gather(hits_v, [sel])
                    pj = jnp.max(plsc.load_gather(hpos_v, [sel]))
                    col = tjv - lo
                    fired = fired_smem[0]

                    @pl.when(fired >= _NROW)
                    def _():
                        pltpu.make_async_copy(
                            rows_v.at[0], stage_hbm.at[0, 0], rsem
                        ).wait()

                    slot = fired & (_NROW - 1)
                    for q in range(EMBED_DIM // _NL):
                        v = plsc.load_gather(
                            chunk_v.at[cb], [q * _NL + lanes, col]
                        )
                        rows_v[slot, pl.ds(q * _NL, _NL)] = v
                    pltpu.async_copy(rows_v.at[slot], stage_hbm.at[pj, 0], rsem)
                    fired_smem[0] = fired + 1
                    return 0

                lax.fori_loop(0, nh, _tok, 0)

            return 0

        lax.fori_loop(0, ngrp, _grp, 0)
        nxt = k + 2

        @pl.when(nxt < nk)
        def _():
            _start_chunk(nxt, cb)

    @pl.loop(0, (_NCHUNK // _NW + 2) // 2)
    def _chunks(k2):
        for b in range(2):
            k = 2 * k2 + b

            @pl.when(k < nk)
            def _():
                _proc_chunk(k, b)

    rem = jnp.minimum(fired_smem[0], _NROW)

    def _drain(i, _):
        pltpu.make_async_copy(rows_v.at[0], stage_hbm.at[0, 0], rsem).wait()
        return 0

    lax.fori_loop(0, rem, _drain, 0)


@jax.jit
def _gather(tokens, tableT):
    mesh = plsc.VectorSubcoreMesh(core_axis_name="c", subcore_axis_name="s")
    stage = pl.kernel(
        _gather_body,
        out_type=jax.ShapeDtypeStruct((BATCH, 8, EMBED_DIM), jnp.float32),
        mesh=mesh,
        scratch_types=[
            pltpu.VMEM((2, _TOKPIECE), jnp.int32),
            pltpu.VMEM((BATCH + _NL,), jnp.int32),
            pltpu.VMEM((BATCH + _NL,), jnp.int32),
            pltpu.VMEM((_NL,), jnp.int32),
            pltpu.VMEM((_NL,), jnp.int32),
            pltpu.VMEM((2, EMBED_DIM, _CHUNK), jnp.float32),
            pltpu.VMEM((_NROW, EMBED_DIM), jnp.float32),
            pltpu.SMEM((1,), jnp.int32),
            pltpu.SMEM((1,), jnp.int32),
            pltpu.SemaphoreType.DMA,
            pltpu.SemaphoreType.DMA,
            pltpu.SemaphoreType.DMA,
        ],
        compiler_params=pltpu.CompilerParams(needs_layout_passes=False),
    )(tokens, tableT)
    return stage[:, 0, :]


def kernel(tokens, values, W):
    del values
    return _gather(tokens, W.T)


# speedup vs baseline: 2.9898x; 1.3934x over previous
import functools

import jax
import jax.numpy as jnp
from jax import lax
from jax.experimental import pallas as pl
from jax.experimental.pallas import tpu as pltpu
from jax.experimental.pallas import tpu_sc as plsc

VOCAB = 1000000
EMBED_DIM = 64
BATCH = 16384

_info = plsc.get_sparse_core_info()
_NC, _NS, _NL = _info.num_cores, _info.num_subcores, _info.num_lanes
_NW = _NC * _NS

_CHUNK = 512
_NCHUNK = 1953
_TAIL_LO = _NCHUNK * _CHUNK
_SHIFT = 9
_TOKPIECE = 2048
_NPIECE = BATCH // _TOKPIECE
_NROW = 16
_ROW_BYTES = EMBED_DIM * 4


def _gather_body(
    tokens_hbm,
    tableT_hbm,
    stage_hbm,
    tokpiece_v,
    my_tok_v,
    my_pos_v,
    hits_v,
    hpos_v,
    chunk_v,
    rows_v,
    cnt_smem,
    fired_smem,
    tsem,
    csem,
    rsem,
):
    wid = lax.axis_index("s") * _NC + lax.axis_index("c")
    cnt_smem[0] = 0
    fired_smem[0] = 0
    lanes = lax.iota(jnp.int32, _NL)

    def _piece_src(p):
        return tokens_hbm.at[pl.ds(p * _TOKPIECE, _TOKPIECE)]

    def _piece(p, pb):
        pltpu.sync_copy(_piece_src(p), tokpiece_v.at[pb])

        def _grp(j, _):
            t = tokpiece_v[pb, pl.ds(j * _NL, _NL)]
            g = lax.shift_right_logical(t, _SHIFT)
            m = (g & (_NW - 1)) == wid
            cnt = cnt_smem[0]
            plsc.store_compressed(my_tok_v.at[pl.ds(cnt, _NL)], t, mask=m)
            pos = p * _TOKPIECE + j * _NL + lanes
            plsc.store_compressed(my_pos_v.at[pl.ds(cnt, _NL)], pos, mask=m)
            cnt_smem[0] = cnt + jnp.sum(m.astype(jnp.int32))
            return 0

        lax.fori_loop(0, _TOKPIECE // _NL, _grp, 0, unroll=4)

    for p in range(_NPIECE):
        _piece(p, 0)

    cnt = cnt_smem[0]
    ngrp = lax.div(cnt + _NL - 1, _NL)

    nk = lax.div(_NCHUNK - 1 - wid, _NW) + 1

    def _descr(g, cb):
        lo = pl.multiple_of(g * _CHUNK, 128)
        return tableT_hbm.at[:, pl.ds(lo, _CHUNK)], chunk_v.at[cb]

    def _start_chunk(k, cb):
        src, dst = _descr(wid + _NW * k, cb)
        pltpu.async_copy(src, dst, csem)

    def _wait_chunk(k, cb):
        src, dst = _descr(wid + _NW * k, cb)
        pltpu.make_async_copy(src, dst, csem).wait()

    def _proc_chunk(k, cb):
        g = wid + _NW * k
        lo = g * _CHUNK
        _wait_chunk(k, cb)

        def _grp(j, _):
            valid = (j * _NL + lanes) < cnt
            t = my_tok_v[pl.ds(j * _NL, _NL)]
            tg = lax.shift_right_logical(t, _SHIFT)
            m = valid & (tg == g)

            @pl.when(jnp.any(m))
            def _():
                plsc.store_compressed(hits_v.at[...], t, mask=m)
                plsc.store_compressed(hpos_v.at[...], my_pos_v[pl.ds(j * _NL, _NL)], mask=m)
                nh = jnp.sum(m.astype(jnp.int32))

                def _tok(j2, _):
                    sel = jnp.full((_NL,), j2, jnp.int32)
                    tjv = plsc.load_gather(hits_v, [sel])
                    pj = jnp.max(plsc.load_gather(hpos_v, [sel]))
                    col = tjv - lo
                    fired = fired_smem[0]

                    @pl.when(fired >= _NROW)
                    def _():
                        pltpu.make_async_copy(
                            rows_v.at[0], stage_hbm.at[0, 0], rsem
                        ).wait()

                    slot = fired & (_NROW - 1)
                    for q in range(EMBED_DIM // _NL):
                        v = plsc.load_gather(
                            chunk_v.at[cb], [q * _NL + lanes, col]
                        )
                        rows_v[slot, pl.ds(q * _NL, _NL)] = v
                    pltpu.async_copy(rows_v.at[slot], stage_hbm.at[pj, 0], rsem)
                    fired_smem[0] = fired + 1
                    return 0

                lax.fori_loop(0, nh, _tok, 0)

            return 0

        lax.fori_loop(0, ngrp, _grp, 0)
        nxt = k + 2

        @pl.when(nxt < nk)
        def _():
            _start_chunk(nxt, cb)

    _start_chunk(0, 0)
    _start_chunk(1, 1)

    @pl.loop(0, (_NCHUNK // _NW + 2) // 2)
    def _chunks(k2):
        for b in range(2):
            k = 2 * k2 + b

            @pl.when(k < nk)
            def _():
                _proc_chunk(k, b)

    rem = jnp.minimum(fired_smem[0], _NROW)

    def _drain(i, _):
        pltpu.make_async_copy(rows_v.at[0], stage_hbm.at[0, 0], rsem).wait()
        return 0

    lax.fori_loop(0, rem, _drain, 0)


@jax.jit
def _gather(tokens, tableT):
    mesh = plsc.VectorSubcoreMesh(core_axis_name="c", subcore_axis_name="s")
    stage = pl.kernel(
        _gather_body,
        out_type=jax.ShapeDtypeStruct((BATCH, 8, EMBED_DIM), jnp.float32),
        mesh=mesh,
        scratch_types=[
            pltpu.VMEM((2, _TOKPIECE), jnp.int32),
            pltpu.VMEM((BATCH + _NL,), jnp.int32),
            pltpu.VMEM((BATCH + _NL,), jnp.int32),
            pltpu.VMEM((_NL,), jnp.int32),
            pltpu.VMEM((_NL,), jnp.int32),
            pltpu.VMEM((2, EMBED_DIM, _CHUNK), jnp.float32),
            pltpu.VMEM((_NROW, EMBED_DIM), jnp.float32),
            pltpu.SMEM((1,), jnp.int32),
            pltpu.SMEM((1,), jnp.int32),
            pltpu.SemaphoreType.DMA,
            pltpu.SemaphoreType.DMA,
            pltpu.SemaphoreType.DMA,
        ],
        compiler_params=pltpu.CompilerParams(needs_layout_passes=False),
    )(tokens, tableT)
    return stage[:, 0, :]


def kernel(tokens, values, W):
    del values
    out = _gather(tokens, W.T)
    tail = jnp.take(W[_TAIL_LO:], jnp.clip(tokens - _TAIL_LO, 0, VOCAB - _TAIL_LO - 1), axis=0)
    return jnp.where((tokens >= _TAIL_LO)[:, None], tail, out)


# speedup vs baseline: 3.1606x; 1.0571x over previous
import functools

import jax
import jax.numpy as jnp
from jax import lax
from jax.experimental import pallas as pl
from jax.experimental.pallas import tpu as pltpu
from jax.experimental.pallas import tpu_sc as plsc

VOCAB = 1000000
EMBED_DIM = 64
BATCH = 16384

_info = plsc.get_sparse_core_info()
_NC, _NS, _NL = _info.num_cores, _info.num_subcores, _info.num_lanes
_NW = _NC * _NS

_CHUNK = 512
_NCHUNK = 1953
_TAIL_LO = _NCHUNK * _CHUNK
_SHIFT = 9
_TOKPIECE = 2048
_NPIECE = BATCH // _TOKPIECE
_NROW = 16
_ROW_BYTES = EMBED_DIM * 4


def _gather_body(
    tokens_hbm,
    tableT_hbm,
    stage_hbm,
    tokpiece_v,
    my_tok_v,
    my_pos_v,
    hits_v,
    hpos_v,
    chunk_v,
    rows_v,
    cnt_smem,
    fired_smem,
    tsem,
    csem,
    rsem,
):
    wid = lax.axis_index("s") * _NC + lax.axis_index("c")
    cnt_smem[0] = 0
    fired_smem[0] = 0
    lanes = lax.iota(jnp.int32, _NL)

    def _piece_src(p):
        return tokens_hbm.at[pl.ds(p * _TOKPIECE, _TOKPIECE)]

    def _piece(p, pb):
        pltpu.sync_copy(_piece_src(p), tokpiece_v.at[pb])

        def _grp(j, _):
            t = tokpiece_v[pb, pl.ds(j * _NL, _NL)]
            g = lax.shift_right_logical(t, _SHIFT)
            m = (g & (_NW - 1)) == wid
            cnt = cnt_smem[0]
            plsc.store_compressed(my_tok_v.at[pl.ds(cnt, _NL)], t, mask=m)
            pos = p * _TOKPIECE + j * _NL + lanes
            plsc.store_compressed(my_pos_v.at[pl.ds(cnt, _NL)], pos, mask=m)
            cnt_smem[0] = cnt + jnp.sum(m.astype(jnp.int32))
            return 0

        lax.fori_loop(0, _TOKPIECE // _NL, _grp, 0, unroll=4)

    for p in range(_NPIECE):
        _piece(p, 0)

    cnt = cnt_smem[0]
    ngrp4 = lax.div(cnt + 4 * _NL - 1, 4 * _NL)

    nk = lax.div(_NCHUNK - 1 - wid, _NW) + 1

    def _descr(g, cb):
        lo = pl.multiple_of(g * _CHUNK, 128)
        return tableT_hbm.at[:, pl.ds(lo, _CHUNK)], chunk_v.at[cb]

    def _start_chunk(k, cb):
        src, dst = _descr(wid + _NW * k, cb)
        pltpu.async_copy(src, dst, csem)

    def _wait_chunk(k, cb):
        src, dst = _descr(wid + _NW * k, cb)
        pltpu.make_async_copy(src, dst, csem).wait()

    def _proc_chunk(k, cb):
        g = wid + _NW * k
        lo = g * _CHUNK
        _wait_chunk(k, cb)

        def _hit_group(j, m, t):
            plsc.store_compressed(hits_v.at[...], t, mask=m)
            plsc.store_compressed(
                hpos_v.at[...], my_pos_v[pl.ds(j * _NL, _NL)], mask=m
            )
            nh = jnp.sum(m.astype(jnp.int32))

            def _tok(j2, _):
                sel = jnp.full((_NL,), j2, jnp.int32)
                tjv = plsc.load_gather(hits_v, [sel])
                pj = jnp.max(plsc.load_gather(hpos_v, [sel]))
                col = tjv - lo
                fired = fired_smem[0]

                @pl.when(fired >= _NROW)
                def _():
                    pltpu.make_async_copy(
                        rows_v.at[0], stage_hbm.at[0, 0], rsem
                    ).wait()

                slot = fired & (_NROW - 1)
                for q in range(EMBED_DIM // _NL):
                    v = plsc.load_gather(
                        chunk_v.at[cb], [q * _NL + lanes, col]
                    )
                    rows_v[slot, pl.ds(q * _NL, _NL)] = v
                pltpu.async_copy(rows_v.at[slot], stage_hbm.at[pj, 0], rsem)
                fired_smem[0] = fired + 1
                return 0

            lax.fori_loop(0, nh, _tok, 0)

        def _grp4(j4, _):
            ts, ms = [], []
            for u in range(4):
                j = j4 * 4 + u
                valid = (j * _NL + lanes) < cnt
                t = my_tok_v[pl.ds(j * _NL, _NL)]
                tg = lax.shift_right_logical(t, _SHIFT)
                ts.append(t)
                ms.append(valid & (tg == g))
            any4 = (ms[0] | ms[1]) | (ms[2] | ms[3])

            @pl.when(jnp.any(any4))
            def _():
                for u in range(4):
                    m, t = ms[u], ts[u]

                    @pl.when(jnp.any(m))
                    def _(j=j4 * 4 + u, m=m, t=t):
                        _hit_group(j, m, t)

            return 0

        lax.fori_loop(0, ngrp4, _grp4, 0)
        nxt = k + 2

        @pl.when(nxt < nk)
        def _():
            _start_chunk(nxt, cb)

    _start_chunk(0, 0)
    _start_chunk(1, 1)

    @pl.loop(0, (_NCHUNK // _NW + 2) // 2)
    def _chunks(k2):
        for b in range(2):
            k = 2 * k2 + b

            @pl.when(k < nk)
            def _():
                _proc_chunk(k, b)

    rem = jnp.minimum(fired_smem[0], _NROW)

    def _drain(i, _):
        pltpu.make_async_copy(rows_v.at[0], stage_hbm.at[0, 0], rsem).wait()
        return 0

    lax.fori_loop(0, rem, _drain, 0)


@jax.jit
def _gather(tokens, tableT):
    mesh = plsc.VectorSubcoreMesh(core_axis_name="c", subcore_axis_name="s")
    stage = pl.kernel(
        _gather_body,
        out_type=jax.ShapeDtypeStruct((BATCH, 8, EMBED_DIM), jnp.float32),
        mesh=mesh,
        scratch_types=[
            pltpu.VMEM((2, _TOKPIECE), jnp.int32),
            pltpu.VMEM((BATCH + 4 * _NL,), jnp.int32),
            pltpu.VMEM((BATCH + 4 * _NL,), jnp.int32),
            pltpu.VMEM((_NL,), jnp.int32),
            pltpu.VMEM((_NL,), jnp.int32),
            pltpu.VMEM((2, EMBED_DIM, _CHUNK), jnp.float32),
            pltpu.VMEM((_NROW, EMBED_DIM), jnp.float32),
            pltpu.SMEM((1,), jnp.int32),
            pltpu.SMEM((1,), jnp.int32),
            pltpu.SemaphoreType.DMA,
            pltpu.SemaphoreType.DMA,
            pltpu.SemaphoreType.DMA,
        ],
        compiler_params=pltpu.CompilerParams(needs_layout_passes=False),
    )(tokens, tableT)
    return stage[:, 0, :]


def kernel(tokens, values, W):
    del values
    out = _gather(tokens, W.T)
    tail = jnp.take(W[_TAIL_LO:], jnp.clip(tokens - _TAIL_LO, 0, VOCAB - _TAIL_LO - 1), axis=0)
    return jnp.where((tokens >= _TAIL_LO)[:, None], tail, out)


# speedup vs baseline: 3.3121x; 1.0479x over previous
import functools

import jax
import jax.numpy as jnp
from jax import lax
from jax.experimental import pallas as pl
from jax.experimental.pallas import tpu as pltpu
from jax.experimental.pallas import tpu_sc as plsc

VOCAB = 1000000
EMBED_DIM = 64
BATCH = 16384

_info = plsc.get_sparse_core_info()
_NC, _NS, _NL = _info.num_cores, _info.num_subcores, _info.num_lanes
_NW = _NC * _NS

_CHUNK = 512
_NCHUNK = 1953
_TAIL_LO = _NCHUNK * _CHUNK
_SHIFT = 9
_TOKPIECE = 2048
_NPIECE = BATCH // _TOKPIECE
_NROW = 16
_ROW_BYTES = EMBED_DIM * 4


def _gather_body(
    tokens_hbm,
    tableT_hbm,
    tailT_hbm,
    stage_hbm,
    tokpiece_v,
    my_tok_v,
    my_pos_v,
    hits_v,
    hpos_v,
    chunk_v,
    tail_v,
    rows_v,
    cnt_smem,
    fired_smem,
    tsem,
    csem,
    rsem,
):
    wid = lax.axis_index("s") * _NC + lax.axis_index("c")
    cnt_smem[0] = 0
    fired_smem[0] = 0
    lanes = lax.iota(jnp.int32, _NL)

    def _piece_src(p):
        return tokens_hbm.at[pl.ds(p * _TOKPIECE, _TOKPIECE)]

    def _piece(p, pb):
        pltpu.sync_copy(_piece_src(p), tokpiece_v.at[pb])

        def _grp(j, _):
            t = tokpiece_v[pb, pl.ds(j * _NL, _NL)]
            g = lax.shift_right_logical(t, _SHIFT)
            m = (g & (_NW - 1)) == wid
            cnt = cnt_smem[0]
            plsc.store_compressed(my_tok_v.at[pl.ds(cnt, _NL)], t, mask=m)
            pos = p * _TOKPIECE + j * _NL + lanes
            plsc.store_compressed(my_pos_v.at[pl.ds(cnt, _NL)], pos, mask=m)
            cnt_smem[0] = cnt + jnp.sum(m.astype(jnp.int32))
            return 0

        lax.fori_loop(0, _TOKPIECE // _NL, _grp, 0, unroll=4)

    for p in range(_NPIECE):
        _piece(p, 0)

    cnt = cnt_smem[0]
    ngrp4 = lax.div(cnt + 4 * _NL - 1, 4 * _NL)

    nk = lax.div(_NCHUNK - 1 - wid, _NW) + 1

    def _descr(g, cb):
        lo = pl.multiple_of(g * _CHUNK, 128)
        return tableT_hbm.at[:, pl.ds(lo, _CHUNK)], chunk_v.at[cb]

    def _start_chunk(k, cb):
        src, dst = _descr(wid + _NW * k, cb)
        pltpu.async_copy(src, dst, csem)

    def _wait_chunk(k, cb):
        src, dst = _descr(wid + _NW * k, cb)
        pltpu.make_async_copy(src, dst, csem).wait()

    def _scan_hits(g, lo, buf_ref):
        def _hit_group(j, m, t):
            plsc.store_compressed(hits_v.at[...], t, mask=m)
            plsc.store_compressed(
                hpos_v.at[...], my_pos_v[pl.ds(j * _NL, _NL)], mask=m
            )
            nh = jnp.sum(m.astype(jnp.int32))

            def _tok(j2, _):
                sel = jnp.full((_NL,), j2, jnp.int32)
                tjv = plsc.load_gather(hits_v, [sel])
                pj = jnp.max(plsc.load_gather(hpos_v, [sel]))
                col = tjv - lo
                fired = fired_smem[0]

                @pl.when(fired >= _NROW)
                def _():
                    pltpu.make_async_copy(
                        rows_v.at[0], stage_hbm.at[0, 0], rsem
                    ).wait()

                slot = fired & (_NROW - 1)
                for q in range(EMBED_DIM // _NL):
                    v = plsc.load_gather(
                        buf_ref, [q * _NL + lanes, col]
                    )
                    rows_v[slot, pl.ds(q * _NL, _NL)] = v
                pltpu.async_copy(rows_v.at[slot], stage_hbm.at[pj, 0], rsem)
                fired_smem[0] = fired + 1
                return 0

            lax.fori_loop(0, nh, _tok, 0)

        def _grp4(j4, _):
            ts, ms = [], []
            for u in range(4):
                j = j4 * 4 + u
                valid = (j * _NL + lanes) < cnt
                t = my_tok_v[pl.ds(j * _NL, _NL)]
                tg = lax.shift_right_logical(t, _SHIFT)
                ts.append(t)
                ms.append(valid & (tg == g))
            any4 = (ms[0] | ms[1]) | (ms[2] | ms[3])

            @pl.when(jnp.any(any4))
            def _():
                for u in range(4):
                    m, t = ms[u], ts[u]

                    @pl.when(jnp.any(m))
                    def _(j=j4 * 4 + u, m=m, t=t):
                        _hit_group(j, m, t)

            return 0

        lax.fori_loop(0, ngrp4, _grp4, 0)

    def _proc_chunk(k, cb):
        g = wid + _NW * k
        _wait_chunk(k, cb)
        _scan_hits(g, g * _CHUNK, chunk_v.at[cb])
        nxt = k + 2

        @pl.when(nxt < nk)
        def _():
            _start_chunk(nxt, cb)

    _start_chunk(0, 0)
    _start_chunk(1, 1)

    @pl.loop(0, (_NCHUNK // _NW + 2) // 2)
    def _chunks(k2):
        for b in range(2):
            k = 2 * k2 + b

            @pl.when(k < nk)
            def _():
                _proc_chunk(k, b)

    @pl.when(wid == (_NCHUNK & (_NW - 1)))
    def _():
        pltpu.sync_copy(tailT_hbm, tail_v)
        _scan_hits(_NCHUNK, _TAIL_LO, tail_v)

    rem = jnp.minimum(fired_smem[0], _NROW)

    def _drain(i, _):
        pltpu.make_async_copy(rows_v.at[0], stage_hbm.at[0, 0], rsem).wait()
        return 0

    lax.fori_loop(0, rem, _drain, 0)


@jax.jit
def _gather(tokens, tableT):
    mesh = plsc.VectorSubcoreMesh(core_axis_name="c", subcore_axis_name="s")
    stage = pl.kernel(
        _gather_body,
        out_type=jax.ShapeDtypeStruct((BATCH, 8, EMBED_DIM), jnp.float32),
        mesh=mesh,
        scratch_types=[
            pltpu.VMEM((2, _TOKPIECE), jnp.int32),
            pltpu.VMEM((BATCH + 4 * _NL,), jnp.int32),
            pltpu.VMEM((BATCH + 4 * _NL,), jnp.int32),
            pltpu.VMEM((_NL,), jnp.int32),
            pltpu.VMEM((_NL,), jnp.int32),
            pltpu.VMEM((2, EMBED_DIM, _CHUNK), jnp.float32),
            pltpu.VMEM((EMBED_DIM, VOCAB - _TAIL_LO), jnp.float32),
            pltpu.VMEM((_NROW, EMBED_DIM), jnp.float32),
            pltpu.SMEM((1,), jnp.int32),
            pltpu.SMEM((1,), jnp.int32),
            pltpu.SemaphoreType.DMA,
            pltpu.SemaphoreType.DMA,
            pltpu.SemaphoreType.DMA,
        ],
        compiler_params=pltpu.CompilerParams(needs_layout_passes=False),
    )(tokens, tableT, tableT[:, _TAIL_LO:])
    return stage[:, 0, :]


def kernel(tokens, values, W):
    del values
    return _gather(tokens, W.T)
